# baseline (speedup 1.0000x reference)
import jax
import jax.numpy as jnp
from jax.experimental import pallas as pl

_PLACEHOLDER_TOKEN = 42
_B, _N, _D = 1024, 77, 768
_RBLK = 1024


def _select_kernel(tok_ref, emb_ref, ph_ref, out_ref):
    mask = tok_ref[...] == _PLACEHOLDER_TOKEN
    out_ref[...] = jnp.where(mask, ph_ref[...], emb_ref[...])


def kernel(tokenized_text, embedded_text, placeholder_embedding):
    rows = _B * _N
    tok2 = tokenized_text.reshape(rows, 1)
    emb2 = embedded_text.reshape(rows, _D)
    out2 = pl.pallas_call(
        _select_kernel,
        grid=(rows // _RBLK,),
        in_specs=[
            pl.BlockSpec((_RBLK, 1), lambda i: (i, 0)),
            pl.BlockSpec((_RBLK, _D), lambda i: (i, 0)),
            pl.BlockSpec((1, _D), lambda i: (0, 0)),
        ],
        out_specs=pl.BlockSpec((_RBLK, _D), lambda i: (i, 0)),
        out_shape=jax.ShapeDtypeStruct((rows, _D), embedded_text.dtype),
    )(tok2, emb2, placeholder_embedding)
    return out2.reshape(_B, _N, _D)


# speedup vs baseline: 1.6601x; 1.6601x over previous
import jax
import jax.numpy as jnp
from jax.experimental import pallas as pl

_PLACEHOLDER_TOKEN = 42
_B, _N, _D = 1024, 77, 768
_BBLK = 8


def _select_kernel(tokT_ref, emb_ref, ph_ref, out_ref):
    maskT = tokT_ref[0] == _PLACEHOLDER_TOKEN
    ph = ph_ref[...]
    for b in range(_BBLK):
        out_ref[b] = jnp.where(maskT[:, b : b + 1], ph, emb_ref[b])


def kernel(tokenized_text, embedded_text, placeholder_embedding):
    tokT = tokenized_text.reshape(_B // _BBLK, _BBLK, _N).transpose(0, 2, 1)
    return pl.pallas_call(
        _select_kernel,
        grid=(_B // _BBLK,),
        in_specs=[
            pl.BlockSpec((1, _N, _BBLK), lambda i: (i, 0, 0)),
            pl.BlockSpec((_BBLK, _N, _D), lambda i: (i, 0, 0)),
            pl.BlockSpec((1, _D), lambda i: (0, 0)),
        ],
        out_specs=pl.BlockSpec((_BBLK, _N, _D), lambda i: (i, 0, 0)),
        out_shape=jax.ShapeDtypeStruct((_B, _N, _D), embedded_text.dtype),
    )(tokT, embedded_text, placeholder_embedding)


# speedup vs baseline: 1.7147x; 1.0329x over previous
import jax
import jax.numpy as jnp
from jax.experimental import pallas as pl

_PLACEHOLDER_TOKEN = 42
_B, _N, _D = 1024, 77, 768
_BBLK = 16


def _select_kernel(tokT_ref, emb_ref, ph_ref, out_ref):
    maskT = tokT_ref[0] == _PLACEHOLDER_TOKEN
    ph = ph_ref[...]
    for b in range(_BBLK):
        out_ref[b] = jnp.where(maskT[:, b : b + 1], ph, emb_ref[b])


def kernel(tokenized_text, embedded_text, placeholder_embedding):
    tokT = tokenized_text.reshape(_B // _BBLK, _BBLK, _N).transpose(0, 2, 1)
    return pl.pallas_call(
        _select_kernel,
        grid=(_B // _BBLK,),
        in_specs=[
            pl.BlockSpec((1, _N, _BBLK), lambda i: (i, 0, 0)),
            pl.BlockSpec((_BBLK, _N, _D), lambda i: (i, 0, 0)),
            pl.BlockSpec((1, _D), lambda i: (0, 0)),
        ],
        out_specs=pl.BlockSpec((_BBLK, _N, _D), lambda i: (i, 0, 0)),
        out_shape=jax.ShapeDtypeStruct((_B, _N, _D), embedded_text.dtype),
    )(tokT, embedded_text, placeholder_embedding)
